# baseline (speedup 1.0000x reference)
import functools
import jax
import jax.numpy as jnp
from jax.experimental import pallas as pl
from jax.experimental.pallas import tpu as pltpu

B, L_V, D = 16, 576, 768
TARGET = 172
PIVOTS = 43
TOPK_PER_PIVOT = 3
_HI = jax.lax.Precision.HIGHEST


def _norms_kernel(hid_ref, out_ref):
    x = hid_ref[0]
    out_ref[...] = jnp.sqrt(jnp.sum(x * x, axis=-1)).reshape(1, 1, L_V)


def _pivot_kernel(norms_ref, piv_ref, pivnorm_ref, s0_ref):
    norms = norms_ref[...].reshape(B, L_V)
    iota_l = jax.lax.broadcasted_iota(jnp.int32, (B, L_V), 1)
    iota_p = jax.lax.broadcasted_iota(jnp.int32, (B, PIVOTS), 1)

    def body(i, carry):
        act, piv, pivnorm = carry
        a = act > 0.5
        v = jnp.where(a, norms, -jnp.inf)
        m = jnp.max(v, axis=1, keepdims=True)
        j = jnp.min(jnp.where((v == m) & a, iota_l, L_V), axis=1, keepdims=True)
        piv = jnp.where(iota_p == i, j, piv)
        pivnorm = jnp.where(iota_p == i, m, pivnorm)
        act = jnp.where(iota_l == j, 0.0, act)
        return act, piv, pivnorm

    act0 = jnp.ones((B, L_V), jnp.float32)
    piv0 = jnp.zeros((B, PIVOTS), jnp.int32)
    pn0 = jnp.zeros((B, PIVOTS), jnp.float32)
    act, piv, pivnorm = jax.lax.fori_loop(0, PIVOTS, body, (act0, piv0, pn0))
    piv_ref[...] = piv.reshape(B, PIVOTS, 1)
    pivnorm_ref[...] = pivnorm.reshape(B, PIVOTS, 1)
    s0_ref[...] = (1.0 - act).reshape(B, 1, L_V)


def _cos_kernel(hid_ref, piv_ref, pivnorm_ref, norms_ref, cos_ref, pv_ref):
    pivnorm = pivnorm_ref[0]
    norms = norms_ref[0]

    def gbody(i, _):
        j = piv_ref[0, i, 0]
        pv_ref[pl.ds(i, 1), :] = hid_ref[0, pl.ds(j, 1), :]
        return 0

    jax.lax.fori_loop(0, PIVOTS, gbody, 0)
    num = jax.lax.dot_general(pv_ref[...], hid_ref[0], (((1,), (1,)), ((), ())),
                              precision=_HI)
    den = jnp.maximum(pivnorm * norms, 1e-8)
    cos_ref[...] = (num / den).reshape(1, PIVOTS, L_V)


def _select_kernel(cos_ref, s0_ref, ridx_ref):
    sel0 = s0_ref[...].reshape(B, L_V)
    act0 = 1.0 - sel0
    iota_l = jax.lax.broadcasted_iota(jnp.int32, (B, L_V), 1)

    def body(i, carry):
        sel, act = carry
        c = cos_ref[pl.ds(i, 1)].reshape(B, L_V)
        for _ in range(TOPK_PER_PIVOT):
            a = act > 0.5
            v = jnp.where(a, c, jnp.inf)
            m = jnp.min(v, axis=1, keepdims=True)
            j = jnp.min(jnp.where((v == m) & a, iota_l, L_V),
                        axis=1, keepdims=True)
            sel = jnp.where(iota_l == j, 1.0, sel)
            act = jnp.where(iota_l == j, 0.0, act)
        return sel, act

    sel, act = jax.lax.fori_loop(0, PIVOTS, body, (sel0, act0))

    lt = (jax.lax.broadcasted_iota(jnp.int32, (L_V, L_V), 0)
          <= jax.lax.broadcasted_iota(jnp.int32, (L_V, L_V), 1))
    pos_incl = jax.lax.dot(sel.astype(jnp.float32), lt.astype(jnp.float32),
                           precision=_HI)
    iota_t = jax.lax.broadcasted_iota(jnp.int32, (B, TARGET), 1)

    def cbody(k, acc):
        cnt = jnp.sum((pos_incl <= k).astype(jnp.float32), axis=1, keepdims=True)
        return jnp.where(iota_t == k, cnt.astype(jnp.int32), acc)

    ridx = jax.lax.fori_loop(0, TARGET, cbody, jnp.zeros((B, TARGET), jnp.int32))
    ridx_ref[...] = ridx.reshape(B, TARGET, 1)


def _gather_kernel(hid_ref, ridx_ref, out_ref):
    def body(k, _):
        j = ridx_ref[0, k, 0]
        out_ref[0, pl.ds(k, 1), :] = hid_ref[0, pl.ds(j, 1), :]
        return 0

    jax.lax.fori_loop(0, TARGET, body, 0)


def kernel(hidden_v, posteriors_v):
    del posteriors_v
    f32 = jnp.float32

    norms = pl.pallas_call(
        _norms_kernel,
        grid=(B,),
        in_specs=[pl.BlockSpec((1, L_V, D), lambda b: (b, 0, 0))],
        out_specs=pl.BlockSpec((1, 1, L_V), lambda b: (b, 0, 0)),
        out_shape=jax.ShapeDtypeStruct((B, 1, L_V), f32),
    )(hidden_v)

    piv, pivnorm, s0 = pl.pallas_call(
        _pivot_kernel,
        in_specs=[pl.BlockSpec((B, 1, L_V), lambda: (0, 0, 0))],
        out_specs=[
            pl.BlockSpec((B, PIVOTS, 1), lambda: (0, 0, 0)),
            pl.BlockSpec((B, PIVOTS, 1), lambda: (0, 0, 0)),
            pl.BlockSpec((B, 1, L_V), lambda: (0, 0, 0)),
        ],
        out_shape=[
            jax.ShapeDtypeStruct((B, PIVOTS, 1), jnp.int32),
            jax.ShapeDtypeStruct((B, PIVOTS, 1), f32),
            jax.ShapeDtypeStruct((B, 1, L_V), f32),
        ],
    )(norms)

    cos = pl.pallas_call(
        _cos_kernel,
        grid=(B,),
        in_specs=[
            pl.BlockSpec((1, L_V, D), lambda b: (b, 0, 0)),
            pl.BlockSpec((1, PIVOTS, 1), lambda b: (b, 0, 0)),
            pl.BlockSpec((1, PIVOTS, 1), lambda b: (b, 0, 0)),
            pl.BlockSpec((1, 1, L_V), lambda b: (b, 0, 0)),
        ],
        out_specs=pl.BlockSpec((1, PIVOTS, L_V), lambda b: (b, 0, 0)),
        out_shape=jax.ShapeDtypeStruct((B, PIVOTS, L_V), f32),
        scratch_shapes=[pltpu.VMEM((PIVOTS, D), f32)],
    )(hidden_v, piv, pivnorm, norms)
    cos = jnp.transpose(cos, (1, 0, 2))

    ridx3 = pl.pallas_call(
        _select_kernel,
        in_specs=[
            pl.BlockSpec((PIVOTS, B, L_V), lambda: (0, 0, 0)),
            pl.BlockSpec((B, 1, L_V), lambda: (0, 0, 0)),
        ],
        out_specs=pl.BlockSpec((B, TARGET, 1), lambda: (0, 0, 0)),
        out_shape=jax.ShapeDtypeStruct((B, TARGET, 1), jnp.int32),
    )(cos, s0)

    pruned = pl.pallas_call(
        _gather_kernel,
        grid=(B,),
        in_specs=[
            pl.BlockSpec((1, L_V, D), lambda b: (b, 0, 0)),
            pl.BlockSpec((1, TARGET, 1), lambda b: (b, 0, 0)),
        ],
        out_specs=pl.BlockSpec((1, TARGET, D), lambda b: (b, 0, 0)),
        out_shape=jax.ShapeDtypeStruct((B, TARGET, D), f32),
    )(hidden_v, ridx3)

    ridx = ridx3.reshape(B, TARGET).astype(jnp.int64)
    mask = jnp.ones((B, TARGET), dtype=bool)
    return (pruned, ridx, mask)


# speedup vs baseline: 1.4370x; 1.4370x over previous
import functools
import jax
import jax.numpy as jnp
from jax.experimental import pallas as pl
from jax.experimental.pallas import tpu as pltpu

B, L_V, D = 16, 576, 768
TARGET = 172
PIVOTS = 43
TOPK_PER_PIVOT = 3
_HI = jax.lax.Precision.HIGHEST


def _norms_kernel(hid_ref, out_ref):
    x = hid_ref[0]
    out_ref[...] = jnp.sqrt(jnp.sum(x * x, axis=-1)).reshape(1, 1, L_V)


def _pivot_kernel(norms_ref, piv_ref, pivnorm_ref, s0_ref):
    norms = norms_ref[...].reshape(B, L_V)
    iota_l = jax.lax.broadcasted_iota(jnp.int32, (B, L_V), 1)
    iota_p = jax.lax.broadcasted_iota(jnp.int32, (B, PIVOTS), 1)

    def body(i, carry):
        act, piv, pivnorm = carry
        a = act > 0.5
        v = jnp.where(a, norms, -jnp.inf)
        m = jnp.max(v, axis=1, keepdims=True)
        j = jnp.min(jnp.where((v == m) & a, iota_l, L_V), axis=1, keepdims=True)
        piv = jnp.where(iota_p == i, j, piv)
        pivnorm = jnp.where(iota_p == i, m, pivnorm)
        act = jnp.where(iota_l == j, 0.0, act)
        return act, piv, pivnorm

    act0 = jnp.ones((B, L_V), jnp.float32)
    piv0 = jnp.zeros((B, PIVOTS), jnp.int32)
    pn0 = jnp.zeros((B, PIVOTS), jnp.float32)
    act, piv, pivnorm = jax.lax.fori_loop(0, PIVOTS, body, (act0, piv0, pn0))
    piv_ref[...] = piv.reshape(B, PIVOTS, 1)
    pivnorm_ref[...] = pivnorm.reshape(B, PIVOTS, 1)
    s0_ref[...] = (1.0 - act).reshape(B, 1, L_V)


def _cos_kernel(hid_ref, piv_ref, pivnorm_ref, norms_ref, cos_ref, pv_ref):
    pivnorm = pivnorm_ref[0]
    norms = norms_ref[0]

    def gbody(i, _):
        j = piv_ref[0, i, 0]
        pv_ref[pl.ds(i, 1), :] = hid_ref[0, pl.ds(j, 1), :]
        return 0

    jax.lax.fori_loop(0, PIVOTS, gbody, 0)
    num = jax.lax.dot_general(pv_ref[...], hid_ref[0], (((1,), (1,)), ((), ())),
                              precision=_HI)
    den = jnp.maximum(pivnorm * norms, 1e-8)
    cos_ref[...] = (num / den).reshape(1, PIVOTS, L_V)


def _select_kernel(cos_ref, s0_ref, ridx_ref):
    sel0 = s0_ref[...].reshape(B, L_V)
    act0 = 1.0 - sel0
    iota_l = jax.lax.broadcasted_iota(jnp.int32, (B, L_V), 1)

    def body(i, carry):
        sel, act = carry
        c = cos_ref[pl.ds(i, 1)].reshape(B, L_V)
        for _ in range(TOPK_PER_PIVOT):
            a = act > 0.5
            v = jnp.where(a, c, jnp.inf)
            m = jnp.min(v, axis=1, keepdims=True)
            j = jnp.min(jnp.where((v == m) & a, iota_l, L_V),
                        axis=1, keepdims=True)
            sel = jnp.where(iota_l == j, 1.0, sel)
            act = jnp.where(iota_l == j, 0.0, act)
        return sel, act

    sel, act = jax.lax.fori_loop(0, PIVOTS, body, (sel0, act0))

    lt = (jax.lax.broadcasted_iota(jnp.int32, (L_V, L_V), 0)
          <= jax.lax.broadcasted_iota(jnp.int32, (L_V, L_V), 1))
    pos_incl = jax.lax.dot(sel.astype(jnp.float32), lt.astype(jnp.float32),
                           precision=_HI)
    iota_t = jax.lax.broadcasted_iota(jnp.int32, (B, TARGET), 1)

    def cbody(k, acc):
        cnt = jnp.sum((pos_incl <= k).astype(jnp.float32), axis=1, keepdims=True)
        return jnp.where(iota_t == k, cnt.astype(jnp.int32), acc)

    ridx = jax.lax.fori_loop(0, TARGET, cbody, jnp.zeros((B, TARGET), jnp.int32))
    ridx_ref[...] = ridx.reshape(B, TARGET, 1)


def _gather_kernel(hid_ref, ridx_ref, out_ref):
    hid = hid_ref[0]
    ridx = ridx_ref[0]
    onehot = (ridx == jax.lax.broadcasted_iota(jnp.int32, (TARGET, L_V), 1))
    out_ref[...] = jax.lax.dot(onehot.astype(jnp.float32), hid,
                               precision=_HI).reshape(1, TARGET, D)


def kernel(hidden_v, posteriors_v):
    del posteriors_v
    f32 = jnp.float32

    norms = pl.pallas_call(
        _norms_kernel,
        grid=(B,),
        in_specs=[pl.BlockSpec((1, L_V, D), lambda b: (b, 0, 0))],
        out_specs=pl.BlockSpec((1, 1, L_V), lambda b: (b, 0, 0)),
        out_shape=jax.ShapeDtypeStruct((B, 1, L_V), f32),
    )(hidden_v)

    piv, pivnorm, s0 = pl.pallas_call(
        _pivot_kernel,
        in_specs=[pl.BlockSpec((B, 1, L_V), lambda: (0, 0, 0))],
        out_specs=[
            pl.BlockSpec((B, PIVOTS, 1), lambda: (0, 0, 0)),
            pl.BlockSpec((B, PIVOTS, 1), lambda: (0, 0, 0)),
            pl.BlockSpec((B, 1, L_V), lambda: (0, 0, 0)),
        ],
        out_shape=[
            jax.ShapeDtypeStruct((B, PIVOTS, 1), jnp.int32),
            jax.ShapeDtypeStruct((B, PIVOTS, 1), f32),
            jax.ShapeDtypeStruct((B, 1, L_V), f32),
        ],
    )(norms)

    cos = pl.pallas_call(
        _cos_kernel,
        grid=(B,),
        in_specs=[
            pl.BlockSpec((1, L_V, D), lambda b: (b, 0, 0)),
            pl.BlockSpec((1, PIVOTS, 1), lambda b: (b, 0, 0)),
            pl.BlockSpec((1, PIVOTS, 1), lambda b: (b, 0, 0)),
            pl.BlockSpec((1, 1, L_V), lambda b: (b, 0, 0)),
        ],
        out_specs=pl.BlockSpec((1, PIVOTS, L_V), lambda b: (b, 0, 0)),
        out_shape=jax.ShapeDtypeStruct((B, PIVOTS, L_V), f32),
        scratch_shapes=[pltpu.VMEM((PIVOTS, D), f32)],
    )(hidden_v, piv, pivnorm, norms)
    cos = jnp.transpose(cos, (1, 0, 2))

    ridx3 = pl.pallas_call(
        _select_kernel,
        in_specs=[
            pl.BlockSpec((PIVOTS, B, L_V), lambda: (0, 0, 0)),
            pl.BlockSpec((B, 1, L_V), lambda: (0, 0, 0)),
        ],
        out_specs=pl.BlockSpec((B, TARGET, 1), lambda: (0, 0, 0)),
        out_shape=jax.ShapeDtypeStruct((B, TARGET, 1), jnp.int32),
    )(cos, s0)

    pruned = pl.pallas_call(
        _gather_kernel,
        grid=(B,),
        in_specs=[
            pl.BlockSpec((1, L_V, D), lambda b: (b, 0, 0)),
            pl.BlockSpec((1, TARGET, 1), lambda b: (b, 0, 0)),
        ],
        out_specs=pl.BlockSpec((1, TARGET, D), lambda b: (b, 0, 0)),
        out_shape=jax.ShapeDtypeStruct((B, TARGET, D), f32),
    )(hidden_v, ridx3)

    ridx = ridx3.reshape(B, TARGET).astype(jnp.int64)
    mask = jnp.ones((B, TARGET), dtype=bool)
    return (pruned, ridx, mask)


# speedup vs baseline: 1.7020x; 1.1844x over previous
import functools
import jax
import jax.numpy as jnp
from jax.experimental import pallas as pl
from jax.experimental.pallas import tpu as pltpu

B, L_V, D = 16, 576, 768
TARGET = 172
PIVOTS = 43
TOPK_PER_PIVOT = 3
_HI = jax.lax.Precision.HIGHEST


def _norms_kernel(hid_ref, out_ref):
    x = hid_ref[0]
    out_ref[...] = jnp.sqrt(jnp.sum(x * x, axis=-1)).reshape(1, 1, L_V)


def _pivot_kernel(norms_ref, piv_ref, pivnorm_ref, s0_ref):
    norms = norms_ref[...].reshape(B, L_V)
    iota_l = jax.lax.broadcasted_iota(jnp.int32, (B, L_V), 1)
    iota_p = jax.lax.broadcasted_iota(jnp.int32, (B, PIVOTS), 1)

    def body(i, carry):
        act, piv, pivnorm = carry
        a = act > 0.5
        v = jnp.where(a, norms, -jnp.inf)
        m = jnp.max(v, axis=1, keepdims=True)
        j = jnp.min(jnp.where((v == m) & a, iota_l, L_V), axis=1, keepdims=True)
        piv = jnp.where(iota_p == i, j, piv)
        pivnorm = jnp.where(iota_p == i, m, pivnorm)
        act = jnp.where(iota_l == j, 0.0, act)
        return act, piv, pivnorm

    act0 = jnp.ones((B, L_V), jnp.float32)
    piv0 = jnp.zeros((B, PIVOTS), jnp.int32)
    pn0 = jnp.zeros((B, PIVOTS), jnp.float32)
    act, piv, pivnorm = jax.lax.fori_loop(0, PIVOTS, body, (act0, piv0, pn0))
    piv_ref[...] = piv.reshape(B, PIVOTS, 1)
    pivnorm_ref[...] = pivnorm.reshape(B, PIVOTS, 1)
    s0_ref[...] = (1.0 - act).reshape(B, 1, L_V)


def _cos_kernel(hid_ref, piv_ref, pivnorm_ref, norms_ref, cos_ref):
    hid = hid_ref[0]
    piv = piv_ref[0]
    pivnorm = pivnorm_ref[0]
    norms = norms_ref[0]
    onehot = (piv == jax.lax.broadcasted_iota(jnp.int32, (PIVOTS, L_V), 1))
    pivvecs = jax.lax.dot(onehot.astype(jnp.float32), hid, precision=_HI)
    num = jax.lax.dot_general(pivvecs, hid, (((1,), (1,)), ((), ())),
                              precision=_HI)
    den = jnp.maximum(pivnorm * norms, 1e-8)
    cos_ref[...] = (num / den).reshape(1, PIVOTS, L_V)


def _select_kernel(cos_ref, s0_ref, ridx_ref):
    sel0 = s0_ref[...].reshape(B, L_V)
    act0 = 1.0 - sel0
    iota_l = jax.lax.broadcasted_iota(jnp.int32, (B, L_V), 1)

    def body(i, carry):
        sel, act = carry
        c = cos_ref[:, pl.ds(i, 1), :].reshape(B, L_V)
        for _ in range(TOPK_PER_PIVOT):
            a = act > 0.5
            v = jnp.where(a, c, jnp.inf)
            m = jnp.min(v, axis=1, keepdims=True)
            j = jnp.min(jnp.where((v == m) & a, iota_l, L_V),
                        axis=1, keepdims=True)
            sel = jnp.where(iota_l == j, 1.0, sel)
            act = jnp.where(iota_l == j, 0.0, act)
        return sel, act

    sel, act = jax.lax.fori_loop(0, PIVOTS, body, (sel0, act0))

    lt = (jax.lax.broadcasted_iota(jnp.int32, (L_V, L_V), 0)
          <= jax.lax.broadcasted_iota(jnp.int32, (L_V, L_V), 1))
    pos_incl = jax.lax.dot(sel.astype(jnp.float32), lt.astype(jnp.float32),
                           precision=_HI)
    iota_k = jax.lax.broadcasted_iota(jnp.int32, (B, TARGET, 1), 1).astype(jnp.float32)
    cmp = (pos_incl.reshape(B, 1, L_V) <= iota_k).astype(jnp.float32)
    ridx = jnp.sum(cmp, axis=2).astype(jnp.int32)
    ridx_ref[...] = ridx.reshape(B, TARGET, 1)


def _gather_kernel(hid_ref, ridx_ref, out_ref):
    hid = hid_ref[0]
    ridx = ridx_ref[0]
    onehot = (ridx == jax.lax.broadcasted_iota(jnp.int32, (TARGET, L_V), 1))
    out_ref[...] = jax.lax.dot(onehot.astype(jnp.float32), hid,
                               precision=_HI).reshape(1, TARGET, D)


def kernel(hidden_v, posteriors_v):
    del posteriors_v
    f32 = jnp.float32

    norms = pl.pallas_call(
        _norms_kernel,
        grid=(B,),
        in_specs=[pl.BlockSpec((1, L_V, D), lambda b: (b, 0, 0))],
        out_specs=pl.BlockSpec((1, 1, L_V), lambda b: (b, 0, 0)),
        out_shape=jax.ShapeDtypeStruct((B, 1, L_V), f32),
    )(hidden_v)

    piv, pivnorm, s0 = pl.pallas_call(
        _pivot_kernel,
        in_specs=[pl.BlockSpec((B, 1, L_V), lambda: (0, 0, 0))],
        out_specs=[
            pl.BlockSpec((B, PIVOTS, 1), lambda: (0, 0, 0)),
            pl.BlockSpec((B, PIVOTS, 1), lambda: (0, 0, 0)),
            pl.BlockSpec((B, 1, L_V), lambda: (0, 0, 0)),
        ],
        out_shape=[
            jax.ShapeDtypeStruct((B, PIVOTS, 1), jnp.int32),
            jax.ShapeDtypeStruct((B, PIVOTS, 1), f32),
            jax.ShapeDtypeStruct((B, 1, L_V), f32),
        ],
    )(norms)

    cos = pl.pallas_call(
        _cos_kernel,
        grid=(B,),
        in_specs=[
            pl.BlockSpec((1, L_V, D), lambda b: (b, 0, 0)),
            pl.BlockSpec((1, PIVOTS, 1), lambda b: (b, 0, 0)),
            pl.BlockSpec((1, PIVOTS, 1), lambda b: (b, 0, 0)),
            pl.BlockSpec((1, 1, L_V), lambda b: (b, 0, 0)),
        ],
        out_specs=pl.BlockSpec((1, PIVOTS, L_V), lambda b: (b, 0, 0)),
        out_shape=jax.ShapeDtypeStruct((B, PIVOTS, L_V), f32),
    )(hidden_v, piv, pivnorm, norms)

    ridx3 = pl.pallas_call(
        _select_kernel,
        in_specs=[
            pl.BlockSpec((B, PIVOTS, L_V), lambda: (0, 0, 0)),
            pl.BlockSpec((B, 1, L_V), lambda: (0, 0, 0)),
        ],
        out_specs=pl.BlockSpec((B, TARGET, 1), lambda: (0, 0, 0)),
        out_shape=jax.ShapeDtypeStruct((B, TARGET, 1), jnp.int32),
    )(cos, s0)

    pruned = pl.pallas_call(
        _gather_kernel,
        grid=(B,),
        in_specs=[
            pl.BlockSpec((1, L_V, D), lambda b: (b, 0, 0)),
            pl.BlockSpec((1, TARGET, 1), lambda b: (b, 0, 0)),
        ],
        out_specs=pl.BlockSpec((1, TARGET, D), lambda b: (b, 0, 0)),
        out_shape=jax.ShapeDtypeStruct((B, TARGET, D), f32),
    )(hidden_v, ridx3)

    ridx = ridx3.reshape(B, TARGET).astype(jnp.int64)
    mask = jnp.ones((B, TARGET), dtype=bool)
    return (pruned, ridx, mask)
